# Initial kernel scaffold; baseline (speedup 1.0000x reference)
import jax
import jax.numpy as jnp
from jax.experimental import pallas as pl


def kernel(feat, centers, text_global, geom_w1, geom_b1, geom_w2, geom_b2, gate_w, gate_b, bias_w, bias_b, edge_w1, edge_b1, edge_w2, edge_b2, msg_w, msg_b, out_w, out_b, ln_g, ln_b):
    raise NotImplementedError("write your pallas kernel here")


# speedup vs baseline: 6.6782x; 6.6782x over previous
import functools

import jax
import jax.numpy as jnp
from jax.experimental import pallas as pl
from jax.experimental.pallas import tpu as pltpu

TILE = 128
KNN = 16


def _knn_kernel(ct_ref, ctT_ref, idx_ref, geom_ref):
    ct = ct_ref[0]
    ctT = ctT_ref[0]
    n = ctT.shape[1]
    cx = ctT[0:1, :]
    cy = ctT[1:2, :]
    cz = ctT[2:3, :]
    sq_j = cx * cx + cy * cy + cz * cz
    tx = ct[:, 0:1]
    ty = ct[:, 1:2]
    tz = ct[:, 2:3]
    sq_i = tx * tx + ty * ty + tz * tz
    dot = jnp.dot(ct, ctT, preferred_element_type=jnp.float32)
    d2 = sq_i + sq_j - 2.0 * dot
    dist = jnp.sqrt(jnp.maximum(d2, 0.0))

    iota = jax.lax.broadcasted_iota(jnp.int32, (TILE, n), 1)
    big = jnp.float32(3.0e38)
    bigi = jnp.int32(2 ** 30)
    idx_cols = []
    geom_parts = []
    d = dist
    for _ in range(KNN):
        m = jnp.min(d, axis=1, keepdims=True)
        j = jnp.min(jnp.where(d == m, iota, bigi), axis=1, keepdims=True)
        sel = iota == j
        gx = jnp.sum(jnp.where(sel, cx, 0.0), axis=1, keepdims=True)
        gy = jnp.sum(jnp.where(sel, cy, 0.0), axis=1, keepdims=True)
        gz = jnp.sum(jnp.where(sel, cz, 0.0), axis=1, keepdims=True)
        d = jnp.where(sel, big, d)
        rx = gx - tx
        ry = gy - ty
        rz = gz - tz
        dd = jnp.sqrt(jnp.maximum(rx * rx + ry * ry + rz * rz, 1e-12)) + 1e-6
        dn = jnp.log1p(dd)
        geom_parts.append(jnp.concatenate([rx, ry, rz, dn], axis=1)[None])
        idx_cols.append(j)
    idx_ref[0] = jnp.concatenate(idx_cols, axis=1)
    geom_ref[0] = jnp.concatenate(geom_parts, axis=0)


def _msg_kernel(feat_ref, w_ref, b_ref, out_ref):
    out_ref[...] = jax.nn.relu(
        jnp.dot(feat_ref[...], w_ref[...],
                preferred_element_type=jnp.float32) + b_ref[...])


def _edge_kernel(feat_t_ref, feat_f_ref, msg_ref, idx_ref, geom_ref, text_ref,
                 gate_w_ref, gate_b_ref, bias_w_ref, bias_b_ref,
                 geom_w1_ref, geom_b1_ref, geom_w2_ref, geom_b2_ref,
                 edge_w1_ref, edge_b1_ref, edge_w2r_ref, edge_b2_ref,
                 out_w1_ref, out_w2_ref, out_b_ref, ln_g_ref, ln_b_ref,
                 o_ref):
    f32 = jnp.float32
    dot = functools.partial(jnp.dot, preferred_element_type=f32)
    feat_i = feat_t_ref[0]
    featF = feat_f_ref[0]
    msgF = msg_ref[0]
    idx = idx_ref[0]
    n = featF.shape[0]
    geom = geom_ref[0].reshape(KNN * TILE, 4)

    tex = text_ref[0]
    tg = jax.nn.sigmoid(dot(tex, gate_w_ref[...]) + gate_b_ref[...])
    tb = dot(tex, bias_w_ref[...]) + bias_b_ref[...]

    g1 = jax.nn.relu(dot(geom, geom_w1_ref[...]) + geom_b1_ref[...])
    gemb = jax.nn.relu(dot(g1, geom_w2_ref[...]) + geom_b2_ref[...])
    gcond = gemb * tg + tb

    iota = jax.lax.broadcasted_iota(jnp.int32, (TILE, n), 1)
    g_parts = [(iota == idx[:, k:k + 1]).astype(f32)[None] for k in range(KNN)]
    G = jnp.concatenate(g_parts, axis=0).reshape(KNN * TILE, n)
    fn = dot(G, featF)

    feat_rep = jnp.concatenate([feat_i] * KNN, axis=0)
    ef = jnp.tanh(feat_rep + fn + gcond)
    h = jax.nn.relu(dot(ef, edge_w1_ref[...]) + edge_b1_ref[...])
    logits = (jnp.sum(h * edge_w2r_ref[...], axis=1, keepdims=True)
              + edge_b2_ref[...])

    lcols = [logits[k * TILE:(k + 1) * TILE, :] for k in range(KNN)]
    lg = jnp.concatenate(lcols, axis=1)
    lmax = jnp.max(lg, axis=1, keepdims=True)
    ex = jnp.exp(lg - lmax)
    alpha = ex / jnp.sum(ex, axis=1, keepdims=True)

    W = jnp.zeros((TILE, n), f32)
    for k in range(KNN):
        W = W + jnp.where(iota == idx[:, k:k + 1], alpha[:, k:k + 1], 0.0)
    ctx = dot(W, msgF)

    out = jax.nn.relu(dot(feat_i, out_w1_ref[...]) + dot(ctx, out_w2_ref[...])
                      + out_b_ref[...])
    x = feat_i + out
    mu = jnp.mean(x, axis=1, keepdims=True)
    var = jnp.mean((x - mu) ** 2, axis=1, keepdims=True)
    o_ref[0] = (x - mu) * jax.lax.rsqrt(var + 1e-5) * ln_g_ref[...] + ln_b_ref[...]


def kernel(feat, centers, text_global, geom_w1, geom_b1, geom_w2, geom_b2,
           gate_w, gate_b, bias_w, bias_b, edge_w1, edge_b1, edge_w2, edge_b2,
           msg_w, msg_b, out_w, out_b, ln_g, ln_b):
    B, N, H = feat.shape
    nt = N // TILE
    f32 = jnp.float32

    centersT = jnp.transpose(centers, (0, 2, 1))
    idx, geomk = pl.pallas_call(
        _knn_kernel,
        grid=(B, nt),
        in_specs=[
            pl.BlockSpec((1, TILE, 3), lambda b, t: (b, t, 0)),
            pl.BlockSpec((1, 3, N), lambda b, t: (b, 0, 0)),
        ],
        out_specs=[
            pl.BlockSpec((1, TILE, KNN), lambda b, t: (b, t, 0)),
            pl.BlockSpec((1, KNN, TILE, 4), lambda b, t: (b, 0, t, 0)),
        ],
        out_shape=[
            jax.ShapeDtypeStruct((B, N, KNN), jnp.int32),
            jax.ShapeDtypeStruct((B, KNN, N, 4), f32),
        ],
        compiler_params=pltpu.CompilerParams(
            dimension_semantics=("parallel", "parallel")),
    )(centers, centersT)

    feat2 = feat.reshape(B * N, H)
    rows = 512
    msg2 = pl.pallas_call(
        _msg_kernel,
        grid=(B * N // rows,),
        in_specs=[
            pl.BlockSpec((rows, H), lambda i: (i, 0)),
            pl.BlockSpec((H, H), lambda i: (0, 0)),
            pl.BlockSpec((1, H), lambda i: (0, 0)),
        ],
        out_specs=pl.BlockSpec((rows, H), lambda i: (i, 0)),
        out_shape=jax.ShapeDtypeStruct((B * N, H), f32),
        compiler_params=pltpu.CompilerParams(
            dimension_semantics=("parallel",)),
    )(feat2, msg_w, msg_b.reshape(1, H))
    msg3 = msg2.reshape(B, N, H)

    Hh = edge_w1.shape[1]
    bcast = lambda b, t: (0, 0)
    w_spec = lambda shape: pl.BlockSpec(shape, bcast)
    out = pl.pallas_call(
        _edge_kernel,
        grid=(B, nt),
        in_specs=[
            pl.BlockSpec((1, TILE, H), lambda b, t: (b, t, 0)),
            pl.BlockSpec((1, N, H), lambda b, t: (b, 0, 0)),
            pl.BlockSpec((1, N, H), lambda b, t: (b, 0, 0)),
            pl.BlockSpec((1, TILE, KNN), lambda b, t: (b, t, 0)),
            pl.BlockSpec((1, KNN, TILE, 4), lambda b, t: (b, 0, t, 0)),
            pl.BlockSpec((1, 1, H), lambda b, t: (b, 0, 0)),
            w_spec((H, H)), w_spec((1, H)),
            w_spec((H, H)), w_spec((1, H)),
            w_spec((4, H)), w_spec((1, H)),
            w_spec((H, H)), w_spec((1, H)),
            w_spec((H, Hh)), w_spec((1, Hh)),
            w_spec((1, Hh)), w_spec((1, 1)),
            w_spec((H, H)), w_spec((H, H)), w_spec((1, H)),
            w_spec((1, H)), w_spec((1, H)),
        ],
        out_specs=pl.BlockSpec((1, TILE, H), lambda b, t: (b, t, 0)),
        out_shape=jax.ShapeDtypeStruct((B, N, H), f32),
        compiler_params=pltpu.CompilerParams(
            dimension_semantics=("parallel", "parallel")),
    )(feat, feat, msg3, idx, geomk, text_global.reshape(B, 1, H),
      gate_w, gate_b.reshape(1, H),
      bias_w, bias_b.reshape(1, H),
      geom_w1, geom_b1.reshape(1, H),
      geom_w2, geom_b2.reshape(1, H),
      edge_w1, edge_b1.reshape(1, Hh),
      edge_w2.reshape(1, Hh), edge_b2.reshape(1, 1),
      out_w[:H], out_w[H:], out_b.reshape(1, H),
      ln_g.reshape(1, H), ln_b.reshape(1, H))
    return out
